# Initial kernel scaffold; baseline (speedup 1.0000x reference)
import jax
import jax.numpy as jnp
from jax.experimental import pallas as pl


def kernel(x, edge_index, edge_weight, edge_attr, Win, b_in, We1, be1, We2, be2, Was, bas, Wad, bad, Wm, bm, Wr, br, Wro, bro):
    raise NotImplementedError("write your pallas kernel here")


# speedup vs baseline: 3.8549x; 3.8549x over previous
import dataclasses
import functools

import jax
import jax.numpy as jnp
from jax import lax
from jax.experimental import pallas as pl
from jax.experimental.pallas import tpu as pltpu
from jax.experimental.pallas import tpu_sc as plsc

N = 10000
E = 160000
D = 256
DE = 16
H = 128
LAYERS = 2

NC = 2
NS = 16
NW = NC * NS
CH = 32
NCH = 160
NBUF = 8
EPW = NCH * CH
EPAD = NW * EPW
NP = 10240
RPT = NP // NS
LQ = H // 16
MW = H // 2

F32 = jnp.float32
BF16 = jnp.bfloat16


def _h0_body(x_ref, w_ref, b_ref, o_ref):
    o_ref[...] = jnp.dot(x_ref[...], w_ref[...],
                         preferred_element_type=F32) + b_ref[...]


def _bias_body(ea_ref, w1_ref, b1_ref, w2c_ref, b2s_ref, o_ref):
    z = jnp.maximum(jnp.dot(ea_ref[...], w1_ref[...],
                            preferred_element_type=F32) + b1_ref[...], 0.0)
    o_ref[...] = jnp.sum(z * w2c_ref[...], axis=1, keepdims=True) + b2s_ref[...]


def _pre_body(h_ref, wm_ref, bm_ref, wsd_ref, bsd_ref, m_ref, a_ref):
    hr = jnp.maximum(h_ref[...], 0.0)
    m_ref[...] = jnp.dot(hr, wm_ref[...],
                         preferred_element_type=F32) + bm_ref[...]
    a_ref[...] = jnp.dot(hr, wsd_ref[...],
                         preferred_element_type=F32) + bsd_ref[...]


def _post_body(h_ref, p_ref, wr_ref, br_ref, o_ref):
    u = (jnp.maximum(h_ref[...], 0.0)
         + p_ref[0].astype(F32) + p_ref[1].astype(F32))
    o_ref[...] = jnp.maximum(
        jnp.dot(u, wr_ref[...], preferred_element_type=F32) + br_ref[...], 0.0)


def _readout_body(h_ref, wro_ref, bro_ref, out_ref, acc_ref):
    i = pl.program_id(0)

    @pl.when(i == 0)
    def _():
        acc_ref[...] = jnp.zeros_like(acc_ref)

    acc_ref[...] += jnp.sum(h_ref[...], axis=0, keepdims=True)

    @pl.when(i == pl.num_programs(0) - 1)
    def _():
        out_ref[...] = (jnp.sum(acc_ref[...] * wro_ref[...], axis=1,
                                keepdims=True) * (1.0 / N) + bro_ref[...])


def _sc_coeff_body(src_h, dst_h, be_h, w_h, as_h, ad_h,
                   coeff_out,
                   src_v, dst_v, be_v, w_v, as_v, ad_v, coeff_v):
    c = lax.axis_index("c")
    s = lax.axis_index("s")
    wid = c * NS + s

    pltpu.sync_copy(src_h.at[wid], src_v)
    pltpu.sync_copy(dst_h.at[wid], dst_v)
    pltpu.sync_copy(be_h.at[wid], be_v)
    pltpu.sync_copy(w_h.at[wid], w_v)
    pltpu.sync_copy(as_h, as_v)
    pltpu.sync_copy(ad_h, ad_v)

    @pl.loop(0, EPW, step=16)
    def _(i):
        a1 = plsc.load_gather(as_v, [src_v[pl.ds(i, 16)]])
        a2 = plsc.load_gather(ad_v, [dst_v[pl.ds(i, 16)]])
        t = a1 + a2 + be_v[pl.ds(i, 16)]
        sg = 1.0 / (1.0 + jnp.exp(-t))
        coeff_v[pl.ds(i, 16)] = sg * w_v[pl.ds(i, 16)]

    pltpu.sync_copy(coeff_v, coeff_out.at[wid])


def _sc_edge_body(src_h, dst_h, coeff_h, m_h,
                  out_h,
                  src_v, dst_v, coeff_v, acc, *rest):
    bufs = rest[:NBUF]
    gsems = rest[NBUF:2 * NBUF]
    ssems = rest[2 * NBUF:3 * NBUF]
    c = lax.axis_index("c")
    s = lax.axis_index("s")
    wid = c * NS + s
    base_r = s * RPT

    pltpu.sync_copy(src_h.at[wid], src_v)
    pltpu.sync_copy(dst_h.at[wid], dst_v)
    pltpu.sync_copy(coeff_h.at[wid], coeff_v)

    b0 = bufs[0]

    @pl.loop(0, CH)
    def _(r):
        for q in range(LQ):
            b0[r, pl.ds(q * 16, 16)] = jnp.zeros((16,), F32)

    for k in range(RPT // CH):
        pltpu.sync_copy(b0, acc.at[pl.ds(base_r + k * CH, CH)])

    plsc.subcore_barrier()

    def scatter_descs(jc, buf):
        descs = []
        for k in range(CH // 16):
            didx = dst_v[pl.ds(jc * CH + k * 16, 16)]
            descs.append((buf.at[pl.ds(k * 16, 16)], acc.at[didx]))
        return descs

    for b in range(NBUF - 1):
        pltpu.async_copy(m_h.at[src_v.at[pl.ds(b * CH, CH)]], bufs[b], gsems[b])

    @pl.loop(0, NCH, step=NBUF)
    def _(j):
        for b in range(NBUF):
            jj = j + b
            buf = bufs[b]
            pltpu.make_async_copy(m_h.at[src_v.at[pl.ds(jj * CH, CH)]],
                                  buf, gsems[b]).wait()
            nj = jj + NBUF - 1
            nb = (b + NBUF - 1) % NBUF

            @pl.when(nj < NCH)
            def _():
                @pl.when(jj >= 1)
                def _():
                    for sref, dref in scatter_descs(jj - 1, bufs[nb]):
                        pltpu.make_async_copy(sref, dref, ssems[nb]).wait()
                pltpu.async_copy(m_h.at[src_v.at[pl.ds(nj * CH, CH)]],
                                 bufs[nb], gsems[nb])

            @pl.loop(0, CH)
            def _(r):
                csp = plsc.load_gather(
                    coeff_v, [jnp.zeros((16,), jnp.int32) + (jj * CH + r)])
                for q in range(LQ):
                    sl = (r, pl.ds(q * 16, 16))
                    buf[sl] = buf[sl] * csp

            for sref, dref in scatter_descs(jj, buf):
                pltpu.async_copy(sref, dref, ssems[b], add=True)

    for b in range(NBUF):
        jt = NCH - NBUF + b
        for sref, dref in scatter_descs(jt, bufs[jt % NBUF]):
            pltpu.make_async_copy(sref, dref, ssems[jt % NBUF]).wait()

    plsc.subcore_barrier()
    pltpu.sync_copy(acc.at[pl.ds(base_r, RPT)],
                    out_h.at[c, pl.ds(base_r, RPT)])


def _sc_compiler_params():
    cp = pltpu.CompilerParams()
    if "needs_layout_passes" in pltpu.CompilerParams.__dataclass_fields__:
        cp = dataclasses.replace(cp, needs_layout_passes=False)
    return cp


def _make_sc_coeff_kernel():
    mesh = plsc.VectorSubcoreMesh(core_axis_name="c", subcore_axis_name="s")
    return pl.kernel(
        _sc_coeff_body,
        out_type=jax.ShapeDtypeStruct((NW, EPW), F32),
        mesh=mesh,
        compiler_params=_sc_compiler_params(),
        scratch_types=[
            pltpu.VMEM((EPW,), jnp.int32),
            pltpu.VMEM((EPW,), jnp.int32),
            pltpu.VMEM((EPW,), F32),
            pltpu.VMEM((EPW,), F32),
            pltpu.VMEM((N,), F32),
            pltpu.VMEM((N,), F32),
            pltpu.VMEM((EPW,), F32),
        ],
    )


def _make_sc_edge_kernel():
    mesh = plsc.VectorSubcoreMesh(core_axis_name="c", subcore_axis_name="s")
    return pl.kernel(
        _sc_edge_body,
        out_type=jax.ShapeDtypeStruct((NC, NP, H), F32),
        mesh=mesh,
        compiler_params=_sc_compiler_params(),
        scratch_types=[
            pltpu.VMEM((EPW,), jnp.int32),
            pltpu.VMEM((EPW,), jnp.int32),
            pltpu.VMEM((EPW,), F32),
            pltpu.VMEM_SHARED((NP, H), F32),
        ] + [pltpu.VMEM((CH, H), F32)] * NBUF
          + [pltpu.SemaphoreType.DMA] * (2 * NBUF),
    )


_NB = 10
_NBK = N // _NB
_EB = 20
_EBK = E // _EB


def kernel(x, edge_index, edge_weight, edge_attr, Win, b_in, We1, be1,
           We2, be2, Was, bas, Wad, bad, Wm, bm, Wr, br, Wro, bro):
    we2c = jnp.sum(We2, axis=1).reshape(1, H)
    wsd = jnp.zeros((H, H), F32).at[:, 0].set(jnp.sum(Was, axis=1)) \
                                 .at[:, 1].set(jnp.sum(Wad, axis=1))
    bsd = jnp.zeros((1, H), F32).at[0, 0].set(jnp.sum(bas)) \
                                 .at[0, 1].set(jnp.sum(bad))
    be2s = jnp.sum(be2).reshape(1, 1)
    wroT = Wro.reshape(1, H)
    bro2 = bro.reshape(1, 1)

    pad = EPAD - E
    src = jnp.pad(edge_index[0], (0, pad)).reshape(NW, EPW)
    dst = jnp.pad(edge_index[1], (0, pad)).reshape(NW, EPW)
    wp = jnp.pad(edge_weight, (0, pad)).reshape(NW, EPW)

    h = pl.pallas_call(
        _h0_body,
        grid=(_NB,),
        in_specs=[pl.BlockSpec((_NBK, D), lambda i: (i, 0)),
                  pl.BlockSpec((D, H), lambda i: (0, 0)),
                  pl.BlockSpec((1, H), lambda i: (0, 0))],
        out_specs=pl.BlockSpec((_NBK, H), lambda i: (i, 0)),
        out_shape=jax.ShapeDtypeStruct((N, H), F32),
    )(x, Win, b_in.reshape(1, H))

    be = pl.pallas_call(
        _bias_body,
        grid=(_EB,),
        in_specs=[pl.BlockSpec((_EBK, DE), lambda i: (i, 0)),
                  pl.BlockSpec((DE, H), lambda i: (0, 0)),
                  pl.BlockSpec((1, H), lambda i: (0, 0)),
                  pl.BlockSpec((1, H), lambda i: (0, 0)),
                  pl.BlockSpec((1, 1), lambda i: (0, 0))],
        out_specs=pl.BlockSpec((_EBK, 1), lambda i: (i, 0)),
        out_shape=jax.ShapeDtypeStruct((E, 1), F32),
    )(edge_attr, We1, be1.reshape(1, H), we2c, be2s)
    bep = jnp.pad(be.reshape(E), (0, pad)).reshape(NW, EPW)

    sc_coeff = _make_sc_coeff_kernel()
    sc_edge = _make_sc_edge_kernel()

    pre = pl.pallas_call(
        _pre_body,
        grid=(_NB,),
        in_specs=[pl.BlockSpec((_NBK, H), lambda i: (i, 0)),
                  pl.BlockSpec((H, H), lambda i: (0, 0)),
                  pl.BlockSpec((1, H), lambda i: (0, 0)),
                  pl.BlockSpec((H, H), lambda i: (0, 0)),
                  pl.BlockSpec((1, H), lambda i: (0, 0))],
        out_specs=[pl.BlockSpec((_NBK, H), lambda i: (i, 0)),
                   pl.BlockSpec((_NBK, H), lambda i: (i, 0))],
        out_shape=[jax.ShapeDtypeStruct((N, H), F32),
                   jax.ShapeDtypeStruct((N, H), F32)],
    )

    post = pl.pallas_call(
        _post_body,
        grid=(_NB,),
        in_specs=[pl.BlockSpec((_NBK, H), lambda i: (i, 0)),
                  pl.BlockSpec((NC, _NBK, H), lambda i: (0, i, 0)),
                  pl.BlockSpec((H, H), lambda i: (0, 0)),
                  pl.BlockSpec((1, H), lambda i: (0, 0))],
        out_specs=pl.BlockSpec((_NBK, H), lambda i: (i, 0)),
        out_shape=jax.ShapeDtypeStruct((N, H), F32),
    )

    readout = pl.pallas_call(
        _readout_body,
        grid=(_NB,),
        in_specs=[pl.BlockSpec((_NBK, H), lambda i: (i, 0)),
                  pl.BlockSpec((1, H), lambda i: (0, 0)),
                  pl.BlockSpec((1, 1), lambda i: (0, 0))],
        out_specs=pl.BlockSpec((1, 1), lambda i: (0, 0)),
        out_shape=jax.ShapeDtypeStruct((1, 1), F32),
        scratch_shapes=[pltpu.VMEM((1, H), F32)],
    )

    bm2 = bm.reshape(1, H)
    br2 = br.reshape(1, H)

    def layer_step(hc, _):
        m, asd = pre(hc, Wm, bm2, wsd, bsd)
        a_s = asd[:, 0]
        a_d = asd[:, 1]
        coeff = sc_coeff(src, dst, bep, wp, a_s, a_d)
        parts = sc_edge(src, dst, coeff, m)
        return post(hc, parts, Wr, br2), None

    h, _ = lax.scan(layer_step, h, None, length=LAYERS)
    out = readout(h, wroT, bro2)
    return out.reshape(())
